# baseline (speedup 1.0000x reference)
import functools

import jax
import jax.numpy as jnp
from jax import lax
from jax.experimental import pallas as pl
from jax.experimental.pallas import tpu as pltpu
from jax.experimental.pallas import tpu_sc as plsc

D_MODEL = 2048
N_HEADS = 16
HEAD_DIM = D_MODEL // N_HEADS
IDX_HEADS = 4
IDX_DIM = 64
NIDX = IDX_HEADS * IDX_DIM
S = 2048
NEG_INF = -1e9

BQ = 256
N_QBLK = S // BQ
WBLK = 512


def _dot(a, b, dims):
    return lax.dot_general(a.astype(jnp.bfloat16), b.astype(jnp.bfloat16),
                           dimension_numbers=(dims, ((), ())),
                           preferred_element_type=jnp.float32)


def _proj_idx_body(h_ref, w_ref, out_ref, pooled_ref):
    i = pl.program_id(0)
    out_ref[...] = _dot(h_ref[...], w_ref[...], ((1,), (0,)))

    part = jnp.sum(h_ref[...].astype(jnp.float32), axis=0,
                   keepdims=True) * (1.0 / S)

    @pl.when(i == 0)
    def _():
        pooled_ref[...] = part

    @pl.when(i > 0)
    def _():
        pooled_ref[...] += part


def _run_proj_idx(h, w_idx):
    bm = 512
    return pl.pallas_call(
        _proj_idx_body,
        grid=(S // bm,),
        in_specs=[
            pl.BlockSpec((bm, D_MODEL), lambda i: (i, 0)),
            pl.BlockSpec((D_MODEL, 2 * NIDX), lambda i: (0, 0)),
        ],
        out_specs=[
            pl.BlockSpec((bm, 2 * NIDX), lambda i: (i, 0)),
            pl.BlockSpec((1, D_MODEL), lambda i: (0, 0)),
        ],
        out_shape=[
            jax.ShapeDtypeStruct((S, 2 * NIDX), jnp.float32),
            jax.ShapeDtypeStruct((1, D_MODEL), jnp.float32),
        ],
    )(h, w_idx)


def _proj_qkv_body(h_ref, w_ref, out_ref):
    out_ref[...] = _dot(h_ref[...], w_ref[...],
                        ((1,), (0,))).astype(jnp.bfloat16)


def _run_proj_qkv(h, w_qkv):
    bm = 512
    n_wblk = w_qkv.shape[1] // WBLK
    return pl.pallas_call(
        _proj_qkv_body,
        grid=(S // bm, n_wblk),
        in_specs=[
            pl.BlockSpec((bm, D_MODEL), lambda i, j: (i, 0)),
            pl.BlockSpec((D_MODEL, WBLK), lambda i, j: (0, j)),
        ],
        out_specs=pl.BlockSpec((bm, WBLK), lambda i, j: (i, j)),
        out_shape=jax.ShapeDtypeStruct((S, w_qkv.shape[1]), jnp.bfloat16),
    )(h, w_qkv)


def _f32_key(x):
    u = lax.bitcast_convert_type(x, jnp.uint32)
    neg = (u >> 31) == jnp.uint32(1)
    return jnp.where(neg, ~u, u | jnp.uint32(0x80000000))


def _scores_body(qi_ref, ki_ref, qb_ref, kb_ref, iw_ref, pooled_ref, cw_ref,
                 cb_ref, scores_ref, k_ref):
    acc = jnp.zeros((BQ, S), jnp.float32)
    for hh in range(IDX_HEADS):
        sl = slice(hh * IDX_DIM, (hh + 1) * IDX_DIM)
        q = qi_ref[:, sl] + qb_ref[:, sl]
        k = ki_ref[:, sl] + kb_ref[:, sl]
        dp = _dot(q, k, ((1,), (1,)))
        acc = acc + iw_ref[hh] * jnp.maximum(dp, 0.0)

    r = _dot(pooled_ref[...], cw_ref[...], ((1,), (0,)))[0, 0] + cb_ref[0]
    ratio = 1.0 / (1.0 + jnp.exp(-r))
    kf = jnp.clip(lax.round(ratio * S, lax.RoundingMethod.TO_NEAREST_EVEN),
                  1.0, float(S))
    kint = kf.astype(jnp.int32)

    scores_ref[...] = lax.bitcast_convert_type(_f32_key(acc), jnp.int32)
    k_ref[...] = jnp.full((1, 128), kint, jnp.int32)


def _run_scores(qi, ki, qb, kb, iw, pooled, cw, cb):
    return pl.pallas_call(
        _scores_body,
        grid=(N_QBLK,),
        in_specs=[
            pl.BlockSpec((BQ, NIDX), lambda i: (i, 0)),
            pl.BlockSpec((S, NIDX), lambda i: (0, 0)),
            pl.BlockSpec((1, NIDX), lambda i: (0, 0)),
            pl.BlockSpec((1, NIDX), lambda i: (0, 0)),
            pl.BlockSpec(memory_space=pltpu.SMEM),
            pl.BlockSpec((1, D_MODEL), lambda i: (0, 0)),
            pl.BlockSpec((D_MODEL, 1), lambda i: (0, 0)),
            pl.BlockSpec(memory_space=pltpu.SMEM),
        ],
        out_specs=[
            pl.BlockSpec((BQ, S), lambda i: (i, 0)),
            pl.BlockSpec((1, 128), lambda i: (0, 0)),
        ],
        out_shape=[
            jax.ShapeDtypeStruct((S, S), jnp.int32),
            jax.ShapeDtypeStruct((1, 128), jnp.int32),
        ],
    )(qi, ki, qb, kb, iw, pooled, cw, cb)


SC_TILES = 32
HALF = S // 2
RPT = HALF // SC_TILES
RB = RPT
NV = S // 16


def _sc_suffix(h):
    return lax.rev(plsc.cumsum(lax.rev(h, (0,))), (0,))


def _sc_select_digit(suffix, krem, suf_v):
    cnt = plsc.all_reduce_population_count(suffix >= krem)
    suf_v[...] = suffix
    above = plsc.load_gather(suf_v, [jnp.minimum(cnt, 15)])
    above = jnp.where(cnt >= 16, 0, above)
    return cnt, krem - above


def _sc_thr_body(keys_hbm, kv_hbm, thr_hbm, keys_v, hist_v,
                 suf_v, kv_v, thr_v):
    c = lax.axis_index("c")
    sb = lax.axis_index("s")
    wid = sb * 2 + c
    base = wid * RPT

    pltpu.sync_copy(kv_hbm, kv_v)

    ones = jnp.ones((16,), jnp.int32)
    zeros16 = jnp.zeros((16,), jnp.int32)
    lanes = lax.iota(jnp.int32, 16)

    for b in range(RPT // RB):
        pltpu.sync_copy(keys_hbm.at[pl.ds((base + b * RB) * S, RB * S)],
                        keys_v)

        def row_fn(r, _):
            rowoff = r * S

            def one_pass(shift, prefix, krem, first):
                for t in range(16):
                    hist_v[pl.ds(t * 16, 16)] = zeros16

                pmask = jnp.int32((1 << (24 - shift)) - 1)
                psh = (prefix >> (shift + 8)) & pmask

                UNROLL = 16

                def scan(j, _):
                    for u in range(UNROLL):
                        x = keys_v[pl.ds(rowoff + (j * UNROLL + u) * 16, 16)]
                        digit = (x >> shift) & jnp.int32(0xFF)
                        if first:
                            plsc.addupdate_scatter(hist_v, [digit], ones)
                        else:
                            act = ((x >> (shift + 8)) & pmask) == psh
                            plsc.addupdate_scatter(hist_v, [digit], ones,
                                                   mask=act)
                    return 0

                lax.fori_loop(0, NV // UNROLL, scan, 0)

                h16 = plsc.load_gather(hist_v, [lanes * 16])
                for t in range(1, 16):
                    h16 = h16 + plsc.load_gather(hist_v, [lanes * 16 + t])

                cnt_hi, krem2 = _sc_select_digit(_sc_suffix(h16),
                                                 krem, suf_v)
                fine = plsc.load_gather(hist_v, [(cnt_hi - 1) * 16 + lanes])
                cnt_lo, krem3 = _sc_select_digit(_sc_suffix(fine),
                                                 krem2, suf_v)
                digit = (cnt_hi - 1) * 16 + (cnt_lo - 1)
                return prefix | (digit << shift), krem3

            prefix = jnp.zeros((16,), jnp.int32)
            krem = kv_v[...]
            prefix, krem = one_pass(24, prefix, krem, True)
            prefix, krem = one_pass(16, prefix, krem, False)
            prefix, krem = one_pass(8, prefix, krem, False)
            prefix, krem = one_pass(0, prefix, krem, False)

            idx = jnp.broadcast_to(b * RB + r, (16,)).astype(jnp.int32)
            plsc.store_scatter(thr_v, [idx], prefix, mask=lanes == 0)
            return 0

        lax.fori_loop(0, RB, row_fn, 0)

    pltpu.sync_copy(thr_v, thr_hbm.at[pl.ds(base, RPT)])


def _run_sc_thr(keys, kvec):
    mesh = plsc.VectorSubcoreMesh(core_axis_name="c", subcore_axis_name="s")
    fn = functools.partial(
        pl.kernel,
        mesh=mesh,
        compiler_params=pltpu.CompilerParams(needs_layout_passes=False),
        out_type=jax.ShapeDtypeStruct((HALF,), jnp.int32),
        scratch_types=[
            pltpu.VMEM((RB * S,), jnp.int32),
            pltpu.VMEM((256,), jnp.int32),
            pltpu.VMEM((16,), jnp.int32),
            pltpu.VMEM((16,), jnp.int32),
            pltpu.VMEM((RPT,), jnp.int32),
        ],
    )(_sc_thr_body)
    return fn(keys.reshape(-1), kvec)


def _run_sc_thr_half(keys, kvec, r0):
    return _run_sc_thr(lax.slice(keys, (r0, 0), (r0 + HALF, S)), kvec)


def _attn_body(q_ref, k_ref, v_ref, keys_ref, thr_ref, out_ref):
    scale = 1.0 / (HEAD_DIM ** 0.5)
    keys = lax.bitcast_convert_type(keys_ref[...], jnp.uint32)
    thr = lax.bitcast_convert_type(thr_ref[...], jnp.uint32)
    add_mask = jnp.where(keys >= thr, 0.0, NEG_INF)
    logits = lax.dot_general(q_ref[...], k_ref[...],
                             dimension_numbers=((((1,), (1,))), ((), ())),
                             preferred_element_type=jnp.float32) * scale
    logits = logits + add_mask
    m = jnp.max(logits, axis=1, keepdims=True)
    p = jnp.exp(logits - m)
    z = jnp.sum(p, axis=1, keepdims=True)
    pv = lax.dot_general(p.astype(jnp.bfloat16), v_ref[...],
                         dimension_numbers=((((1,), (0,))), ((), ())),
                         preferred_element_type=jnp.float32)
    out_ref[...] = (pv * (1.0 / z)).astype(jnp.bfloat16)


def _run_attn(q, k, v, scores, thr):
    nq = q.shape[0] // BQ
    return pl.pallas_call(
        _attn_body,
        grid=(nq, N_HEADS),
        in_specs=[
            pl.BlockSpec((BQ, HEAD_DIM), lambda i, hh: (i, hh)),
            pl.BlockSpec((S, HEAD_DIM), lambda i, hh: (0, hh)),
            pl.BlockSpec((S, HEAD_DIM), lambda i, hh: (0, hh)),
            pl.BlockSpec((BQ, S), lambda i, hh: (i, 0)),
            pl.BlockSpec((BQ, 1), lambda i, hh: (i, 0)),
        ],
        out_specs=pl.BlockSpec((BQ, HEAD_DIM), lambda i, hh: (i, hh)),
        out_shape=jax.ShapeDtypeStruct((q.shape[0], D_MODEL), jnp.bfloat16),
    )(q, k, v, scores, thr)


def _oproj_body(x_ref, w_ref, out_ref):
    out_ref[...] = lax.dot_general(x_ref[...], w_ref[...],
                                   dimension_numbers=((((1,), (0,))), ((), ())),
                                   preferred_element_type=jnp.float32)


def _run_oproj(x, w):
    return pl.pallas_call(
        _oproj_body,
        grid=(N_QBLK,),
        in_specs=[
            pl.BlockSpec((BQ, D_MODEL), lambda i: (i, 0)),
            pl.BlockSpec((D_MODEL, D_MODEL), lambda i: (0, 0)),
        ],
        out_specs=pl.BlockSpec((BQ, D_MODEL), lambda i: (i, 0)),
        out_shape=jax.ShapeDtypeStruct((S, D_MODEL), jnp.float32),
    )(x, w)


def kernel(hidden_states, q_idx_W, q_idx_b, k_idx_W, k_idx_b, idx_weights,
           ctrl_W, ctrl_b, Wq, Wk, Wv, Wo):
    h16 = hidden_states[0].astype(jnp.bfloat16)

    w_idx = jnp.concatenate([q_idx_W, k_idx_W], axis=1).astype(jnp.bfloat16)
    w_qkv = jnp.concatenate([Wq, Wk, Wv], axis=1).astype(jnp.bfloat16)

    proj_idx, pooled = _run_proj_idx(h16, w_idx)
    qkv = _run_proj_qkv(h16, w_qkv)

    qi = proj_idx[:, :NIDX]
    ki = proj_idx[:, NIDX:]
    q = qkv[:, :D_MODEL]
    k = qkv[:, D_MODEL:2 * D_MODEL]
    v = qkv[:, 2 * D_MODEL:]

    keys, kvec = _run_scores(qi, ki, q_idx_b[None, :], k_idx_b[None, :],
                             idx_weights, pooled, ctrl_W, ctrl_b)

    kv16 = kvec[0, :16]
    thr0 = _run_sc_thr_half(keys, kv16, 0)
    thr1 = _run_sc_thr_half(keys, kv16, HALF)
    attn0 = _run_attn(q[:HALF], k, v, keys[:HALF], thr0[:, None])
    attn1 = _run_attn(q[HALF:], k, v, keys[HALF:], thr1[:, None])
    attn = jnp.concatenate([attn0, attn1], axis=0)
    out = _run_oproj(attn, Wo.astype(jnp.bfloat16))
    return out[None]


# speedup vs baseline: 1.1277x; 1.1277x over previous
import functools

import jax
import jax.numpy as jnp
from jax import lax
from jax.experimental import pallas as pl
from jax.experimental.pallas import tpu as pltpu
from jax.experimental.pallas import tpu_sc as plsc

D_MODEL = 2048
N_HEADS = 16
HEAD_DIM = D_MODEL // N_HEADS
IDX_HEADS = 4
IDX_DIM = 64
NIDX = IDX_HEADS * IDX_DIM
S = 2048
NEG_INF = -1e9

BQ = 256
N_QBLK = S // BQ
WBLK = 512


def _dot(a, b, dims):
    return lax.dot_general(a.astype(jnp.bfloat16), b.astype(jnp.bfloat16),
                           dimension_numbers=(dims, ((), ())),
                           preferred_element_type=jnp.float32)


def _proj_idx_body(h_ref, w_ref, out_ref, pooled_ref):
    i = pl.program_id(0)
    out_ref[...] = _dot(h_ref[...], w_ref[...], ((1,), (0,)))

    part = jnp.sum(h_ref[...].astype(jnp.float32), axis=0,
                   keepdims=True) * (1.0 / S)

    @pl.when(i == 0)
    def _():
        pooled_ref[...] = part

    @pl.when(i > 0)
    def _():
        pooled_ref[...] += part


def _run_proj_idx(h, w_idx):
    bm = 512
    return pl.pallas_call(
        _proj_idx_body,
        grid=(S // bm,),
        in_specs=[
            pl.BlockSpec((bm, D_MODEL), lambda i: (i, 0)),
            pl.BlockSpec((D_MODEL, 2 * NIDX), lambda i: (0, 0)),
        ],
        out_specs=[
            pl.BlockSpec((bm, 2 * NIDX), lambda i: (i, 0)),
            pl.BlockSpec((1, D_MODEL), lambda i: (0, 0)),
        ],
        out_shape=[
            jax.ShapeDtypeStruct((S, 2 * NIDX), jnp.float32),
            jax.ShapeDtypeStruct((1, D_MODEL), jnp.float32),
        ],
    )(h, w_idx)


def _proj_qkv_body(h_ref, w_ref, out_ref):
    out_ref[...] = _dot(h_ref[...], w_ref[...],
                        ((1,), (0,))).astype(jnp.bfloat16)


def _run_proj_qkv(h, w_qkv):
    bm = 512
    n_wblk = w_qkv.shape[1] // WBLK
    return pl.pallas_call(
        _proj_qkv_body,
        grid=(S // bm, n_wblk),
        in_specs=[
            pl.BlockSpec((bm, D_MODEL), lambda i, j: (i, 0)),
            pl.BlockSpec((D_MODEL, WBLK), lambda i, j: (0, j)),
        ],
        out_specs=pl.BlockSpec((bm, WBLK), lambda i, j: (i, j)),
        out_shape=jax.ShapeDtypeStruct((S, w_qkv.shape[1]), jnp.bfloat16),
    )(h, w_qkv)


def _f32_key(x):
    u = lax.bitcast_convert_type(x, jnp.uint32)
    neg = (u >> 31) == jnp.uint32(1)
    return jnp.where(neg, ~u, u | jnp.uint32(0x80000000))


def _scores_body(qi_ref, ki_ref, qb_ref, kb_ref, iw_ref, pooled_ref, cw_ref,
                 cb_ref, scores_ref, k_ref):
    acc = jnp.zeros((BQ, S), jnp.float32)
    for hh in range(IDX_HEADS):
        sl = slice(hh * IDX_DIM, (hh + 1) * IDX_DIM)
        q = qi_ref[:, sl] + qb_ref[:, sl]
        k = ki_ref[:, sl] + kb_ref[:, sl]
        dp = _dot(q, k, ((1,), (1,)))
        acc = acc + iw_ref[hh] * jnp.maximum(dp, 0.0)

    r = _dot(pooled_ref[...], cw_ref[...], ((1,), (0,)))[0, 0] + cb_ref[0]
    ratio = 1.0 / (1.0 + jnp.exp(-r))
    kf = jnp.clip(lax.round(ratio * S, lax.RoundingMethod.TO_NEAREST_EVEN),
                  1.0, float(S))
    kint = kf.astype(jnp.int32)

    scores_ref[...] = lax.bitcast_convert_type(_f32_key(acc), jnp.int32)
    k_ref[...] = jnp.full((1, 128), kint, jnp.int32)


def _run_scores(qi, ki, qb, kb, iw, pooled, cw, cb):
    return pl.pallas_call(
        _scores_body,
        grid=(N_QBLK,),
        in_specs=[
            pl.BlockSpec((BQ, NIDX), lambda i: (i, 0)),
            pl.BlockSpec((S, NIDX), lambda i: (0, 0)),
            pl.BlockSpec((1, NIDX), lambda i: (0, 0)),
            pl.BlockSpec((1, NIDX), lambda i: (0, 0)),
            pl.BlockSpec(memory_space=pltpu.SMEM),
            pl.BlockSpec((1, D_MODEL), lambda i: (0, 0)),
            pl.BlockSpec((D_MODEL, 1), lambda i: (0, 0)),
            pl.BlockSpec(memory_space=pltpu.SMEM),
        ],
        out_specs=[
            pl.BlockSpec((BQ, S), lambda i: (i, 0)),
            pl.BlockSpec((1, 128), lambda i: (0, 0)),
        ],
        out_shape=[
            jax.ShapeDtypeStruct((S, S), jnp.int32),
            jax.ShapeDtypeStruct((1, 128), jnp.int32),
        ],
    )(qi, ki, qb, kb, iw, pooled, cw, cb)


SC_TILES = 32
HALF = S // 2
RPT = HALF // SC_TILES
RB = RPT
NV = S // 16


def _sc_suffix(h):
    return lax.rev(plsc.cumsum(lax.rev(h, (0,))), (0,))


def _sc_select_digit(suffix, krem, suf_v):
    cnt = plsc.all_reduce_population_count(suffix >= krem)
    suf_v[...] = suffix
    above = plsc.load_gather(suf_v, [jnp.minimum(cnt, 15)])
    above = jnp.where(cnt >= 16, 0, above)
    return cnt, krem - above


def _sc_thr_body(keys_hbm, kv_hbm, thr_hbm, keys_v, hist_v,
                 suf_v, kv_v, thr_v, ba_v, bb_v):
    c = lax.axis_index("c")
    sb = lax.axis_index("s")
    wid = sb * 2 + c
    base = wid * RPT

    pltpu.sync_copy(kv_hbm, kv_v)
    ktest = lax.squeeze(lax.slice(kv_v[...], (0,), (1,)), dimensions=(0,))

    ones = jnp.ones((16,), jnp.int32)
    zeros16 = jnp.zeros((16,), jnp.int32)
    lanes = lax.iota(jnp.int32, 16)

    for b in range(RPT // RB):
        pltpu.sync_copy(keys_hbm.at[pl.ds((base + b * RB) * S, RB * S)],
                        keys_v)

        def row_fn(r, _):
            rowoff = r * S
            UNROLL = 16

            def zero_hist():
                for t in range(16):
                    hist_v[pl.ds(t * 16, 16)] = zeros16

            def select_byte(krem):
                h16 = plsc.load_gather(hist_v, [lanes * 16])
                for t in range(1, 16):
                    h16 = h16 + plsc.load_gather(hist_v, [lanes * 16 + t])
                cnt_hi, krem2 = _sc_select_digit(_sc_suffix(h16), krem, suf_v)
                fine = plsc.load_gather(hist_v, [(cnt_hi - 1) * 16 + lanes])
                cnt_lo, krem3 = _sc_select_digit(_sc_suffix(fine), krem2,
                                                 suf_v)
                return (cnt_hi - 1) * 16 + (cnt_lo - 1), krem3

            zero_hist()

            def scan1(j, _):
                for u in range(UNROLL):
                    x = keys_v[pl.ds(rowoff + (j * UNROLL + u) * 16, 16)]
                    plsc.addupdate_scatter(
                        hist_v, [(x >> 24) & jnp.int32(0xFF)], ones)
                return 0

            lax.fori_loop(0, NV // UNROLL, scan1, 0)
            d1, krem = select_byte(jnp.broadcast_to(ktest, (16,)))
            prefix = d1 << 24

            def comp1(j, off):
                for u in range(UNROLL):
                    x = keys_v[pl.ds(rowoff + (j * UNROLL + u) * 16, 16)]
                    act = ((x >> 24) & jnp.int32(0xFF)) == d1
                    pos = plsc.cumsum(act.astype(jnp.int32))
                    plsc.store_scatter(ba_v, [off + pos - 1], x, mask=act)
                    off = off + plsc.all_reduce_population_count(act)
                return off

            off = lax.fori_loop(0, NV // UNROLL, comp1, zeros16)
            n = lax.squeeze(lax.slice(off, (0,), (1,)), dimensions=(0,))

            for pi, shift in enumerate((16, 8, 0)):
                src = (ba_v, bb_v)[pi % 2]
                dst = (bb_v, ba_v)[pi % 2]
                nsp = jnp.broadcast_to(n, (16,))
                nv = (n + 15) >> 4
                zero_hist()

                def scanp(j, _, src=src, shift=shift, nsp=nsp):
                    x = src[pl.ds(j * 16, 16)]
                    tail = (j * 16 + lanes) < nsp
                    plsc.addupdate_scatter(
                        hist_v, [(x >> shift) & jnp.int32(0xFF)], ones,
                        mask=tail)
                    return 0

                lax.fori_loop(0, nv, scanp, 0)
                dsub, krem = select_byte(krem)
                prefix = prefix | (dsub << shift)

                if shift != 0:
                    def compp(j, off2, src=src, dst=dst, shift=shift,
                              nsp=nsp, dsub=dsub):
                        x = src[pl.ds(j * 16, 16)]
                        tail = (j * 16 + lanes) < nsp
                        act = tail & (((x >> shift) & jnp.int32(0xFF)) == dsub)
                        pos = plsc.cumsum(act.astype(jnp.int32))
                        plsc.store_scatter(dst, [off2 + pos - 1], x, mask=act)
                        return off2 + plsc.all_reduce_population_count(act)

                    off2 = lax.fori_loop(0, nv, compp, zeros16)
                    n = lax.squeeze(lax.slice(off2, (0,), (1,)),
                                    dimensions=(0,))

            idx = jnp.broadcast_to(b * RB + r, (16,)).astype(jnp.int32)
            plsc.store_scatter(thr_v, [idx], prefix, mask=lanes == 0)
            return 0

        lax.fori_loop(0, RB, row_fn, 0)

    pltpu.sync_copy(thr_v, thr_hbm.at[pl.ds(base, RPT)])


def _run_sc_thr(keys, kvec):
    mesh = plsc.VectorSubcoreMesh(core_axis_name="c", subcore_axis_name="s")
    fn = functools.partial(
        pl.kernel,
        mesh=mesh,
        compiler_params=pltpu.CompilerParams(needs_layout_passes=False),
        out_type=jax.ShapeDtypeStruct((HALF,), jnp.int32),
        scratch_types=[
            pltpu.VMEM((RB * S,), jnp.int32),
            pltpu.VMEM((256,), jnp.int32),
            pltpu.VMEM((16,), jnp.int32),
            pltpu.VMEM((16,), jnp.int32),
            pltpu.VMEM((RPT,), jnp.int32),
            pltpu.VMEM((S,), jnp.int32),
            pltpu.VMEM((S,), jnp.int32),
        ],
    )(_sc_thr_body)
    return fn(keys.reshape(-1), kvec)


def _run_sc_thr_half(keys, kvec, r0):
    return _run_sc_thr(lax.slice(keys, (r0, 0), (r0 + HALF, S)), kvec)


def _attn_body(q_ref, k_ref, v_ref, keys_ref, thr_ref, out_ref):
    scale = 1.0 / (HEAD_DIM ** 0.5)
    keys = lax.bitcast_convert_type(keys_ref[...], jnp.uint32)
    thr = lax.bitcast_convert_type(thr_ref[...], jnp.uint32)
    add_mask = jnp.where(keys >= thr, 0.0, NEG_INF)
    logits = lax.dot_general(q_ref[...], k_ref[...],
                             dimension_numbers=((((1,), (1,))), ((), ())),
                             preferred_element_type=jnp.float32) * scale
    logits = logits + add_mask
    m = jnp.max(logits, axis=1, keepdims=True)
    p = jnp.exp(logits - m)
    z = jnp.sum(p, axis=1, keepdims=True)
    pv = lax.dot_general(p.astype(jnp.bfloat16), v_ref[...],
                         dimension_numbers=((((1,), (0,))), ((), ())),
                         preferred_element_type=jnp.float32)
    out_ref[...] = (pv * (1.0 / z)).astype(jnp.bfloat16)


def _run_attn(q, k, v, scores, thr):
    nq = q.shape[0] // BQ
    return pl.pallas_call(
        _attn_body,
        grid=(nq, N_HEADS),
        in_specs=[
            pl.BlockSpec((BQ, HEAD_DIM), lambda i, hh: (i, hh)),
            pl.BlockSpec((S, HEAD_DIM), lambda i, hh: (0, hh)),
            pl.BlockSpec((S, HEAD_DIM), lambda i, hh: (0, hh)),
            pl.BlockSpec((BQ, S), lambda i, hh: (i, 0)),
            pl.BlockSpec((BQ, 1), lambda i, hh: (i, 0)),
        ],
        out_specs=pl.BlockSpec((BQ, HEAD_DIM), lambda i, hh: (i, hh)),
        out_shape=jax.ShapeDtypeStruct((q.shape[0], D_MODEL), jnp.bfloat16),
    )(q, k, v, scores, thr)


def _oproj_body(x_ref, w_ref, out_ref):
    out_ref[...] = lax.dot_general(x_ref[...], w_ref[...],
                                   dimension_numbers=((((1,), (0,))), ((), ())),
                                   preferred_element_type=jnp.float32)


def _run_oproj(x, w):
    return pl.pallas_call(
        _oproj_body,
        grid=(N_QBLK,),
        in_specs=[
            pl.BlockSpec((BQ, D_MODEL), lambda i: (i, 0)),
            pl.BlockSpec((D_MODEL, D_MODEL), lambda i: (0, 0)),
        ],
        out_specs=pl.BlockSpec((BQ, D_MODEL), lambda i: (i, 0)),
        out_shape=jax.ShapeDtypeStruct((S, D_MODEL), jnp.float32),
    )(x, w)


def kernel(hidden_states, q_idx_W, q_idx_b, k_idx_W, k_idx_b, idx_weights,
           ctrl_W, ctrl_b, Wq, Wk, Wv, Wo):
    h16 = hidden_states[0].astype(jnp.bfloat16)

    w_idx = jnp.concatenate([q_idx_W, k_idx_W], axis=1).astype(jnp.bfloat16)
    w_qkv = jnp.concatenate([Wq, Wk, Wv], axis=1).astype(jnp.bfloat16)

    proj_idx, pooled = _run_proj_idx(h16, w_idx)
    qkv = _run_proj_qkv(h16, w_qkv)

    qi = proj_idx[:, :NIDX]
    ki = proj_idx[:, NIDX:]
    q = qkv[:, :D_MODEL]
    k = qkv[:, D_MODEL:2 * D_MODEL]
    v = qkv[:, 2 * D_MODEL:]

    keys, kvec = _run_scores(qi, ki, q_idx_b[None, :], k_idx_b[None, :],
                             idx_weights, pooled, ctrl_W, ctrl_b)

    kv16 = kvec[0, :16]
    thr0 = _run_sc_thr_half(keys, kv16, 0)
    thr1 = _run_sc_thr_half(keys, kv16, HALF)
    attn0 = _run_attn(q[:HALF], k, v, keys[:HALF], thr0[:, None])
    attn1 = _run_attn(q[HALF:], k, v, keys[HALF:], thr1[:, None])
    attn = jnp.concatenate([attn0, attn1], axis=0)
    out = _run_oproj(attn, Wo.astype(jnp.bfloat16))
    return out[None]


# speedup vs baseline: 1.1682x; 1.0359x over previous
import functools

import jax
import jax.numpy as jnp
from jax import lax
from jax.experimental import pallas as pl
from jax.experimental.pallas import tpu as pltpu
from jax.experimental.pallas import tpu_sc as plsc

D_MODEL = 2048
N_HEADS = 16
HEAD_DIM = D_MODEL // N_HEADS
IDX_HEADS = 4
IDX_DIM = 64
NIDX = IDX_HEADS * IDX_DIM
S = 2048
NEG_INF = -1e9

BQ = 256
N_QBLK = S // BQ
WBLK = 512


def _dot(a, b, dims):
    return lax.dot_general(a.astype(jnp.bfloat16), b.astype(jnp.bfloat16),
                           dimension_numbers=(dims, ((), ())),
                           preferred_element_type=jnp.float32)


def _proj_idx_body(h_ref, w_ref, out_ref, pooled_ref):
    i = pl.program_id(0)
    out_ref[...] = _dot(h_ref[...], w_ref[...], ((1,), (0,)))

    part = jnp.sum(h_ref[...].astype(jnp.float32), axis=0,
                   keepdims=True) * (1.0 / S)

    @pl.when(i == 0)
    def _():
        pooled_ref[...] = part

    @pl.when(i > 0)
    def _():
        pooled_ref[...] += part


def _run_proj_idx(h, w_idx):
    bm = 512
    return pl.pallas_call(
        _proj_idx_body,
        grid=(S // bm,),
        in_specs=[
            pl.BlockSpec((bm, D_MODEL), lambda i: (i, 0)),
            pl.BlockSpec((D_MODEL, 2 * NIDX), lambda i: (0, 0)),
        ],
        out_specs=[
            pl.BlockSpec((bm, 2 * NIDX), lambda i: (i, 0)),
            pl.BlockSpec((1, D_MODEL), lambda i: (0, 0)),
        ],
        out_shape=[
            jax.ShapeDtypeStruct((S, 2 * NIDX), jnp.float32),
            jax.ShapeDtypeStruct((1, D_MODEL), jnp.float32),
        ],
    )(h, w_idx)


def _proj_qkv_body(h_ref, w_ref, out_ref):
    out_ref[...] = _dot(h_ref[...], w_ref[...],
                        ((1,), (0,))).astype(jnp.bfloat16)


def _run_proj_qkv(h, w_qkv):
    bm = 512
    n_wblk = w_qkv.shape[1] // WBLK
    return pl.pallas_call(
        _proj_qkv_body,
        grid=(S // bm, n_wblk),
        in_specs=[
            pl.BlockSpec((bm, D_MODEL), lambda i, j: (i, 0)),
            pl.BlockSpec((D_MODEL, WBLK), lambda i, j: (0, j)),
        ],
        out_specs=pl.BlockSpec((bm, WBLK), lambda i, j: (i, j)),
        out_shape=jax.ShapeDtypeStruct((S, w_qkv.shape[1]), jnp.bfloat16),
    )(h, w_qkv)


def _f32_key(x):
    u = lax.bitcast_convert_type(x, jnp.uint32)
    neg = (u >> 31) == jnp.uint32(1)
    return jnp.where(neg, ~u, u | jnp.uint32(0x80000000))


def _scores_body(qi_ref, ki_ref, qb_ref, kb_ref, iw_ref, pooled_ref, cw_ref,
                 cb_ref, scores_ref, k_ref):
    acc = jnp.zeros((BQ, S), jnp.float32)
    for hh in range(IDX_HEADS):
        sl = slice(hh * IDX_DIM, (hh + 1) * IDX_DIM)
        q = qi_ref[:, sl] + qb_ref[:, sl]
        k = ki_ref[:, sl] + kb_ref[:, sl]
        dp = _dot(q, k, ((1,), (1,)))
        acc = acc + iw_ref[hh] * jnp.maximum(dp, 0.0)

    r = _dot(pooled_ref[...], cw_ref[...], ((1,), (0,)))[0, 0] + cb_ref[0]
    ratio = 1.0 / (1.0 + jnp.exp(-r))
    kf = jnp.clip(lax.round(ratio * S, lax.RoundingMethod.TO_NEAREST_EVEN),
                  1.0, float(S))
    kint = kf.astype(jnp.int32)

    scores_ref[...] = lax.bitcast_convert_type(_f32_key(acc), jnp.int32)
    k_ref[...] = jnp.full((1, 128), kint, jnp.int32)


def _run_scores(qi, ki, qb, kb, iw, pooled, cw, cb):
    return pl.pallas_call(
        _scores_body,
        grid=(N_QBLK,),
        in_specs=[
            pl.BlockSpec((BQ, NIDX), lambda i: (i, 0)),
            pl.BlockSpec((S, NIDX), lambda i: (0, 0)),
            pl.BlockSpec((1, NIDX), lambda i: (0, 0)),
            pl.BlockSpec((1, NIDX), lambda i: (0, 0)),
            pl.BlockSpec(memory_space=pltpu.SMEM),
            pl.BlockSpec((1, D_MODEL), lambda i: (0, 0)),
            pl.BlockSpec((D_MODEL, 1), lambda i: (0, 0)),
            pl.BlockSpec(memory_space=pltpu.SMEM),
        ],
        out_specs=[
            pl.BlockSpec((BQ, S), lambda i: (i, 0)),
            pl.BlockSpec((1, 128), lambda i: (0, 0)),
        ],
        out_shape=[
            jax.ShapeDtypeStruct((S, S), jnp.int32),
            jax.ShapeDtypeStruct((1, 128), jnp.int32),
        ],
    )(qi, ki, qb, kb, iw, pooled, cw, cb)


SC_TILES = 32
RPT = S // SC_TILES
RB = 32
NV = S // 16


def _sc_suffix(h):
    return lax.rev(plsc.cumsum(lax.rev(h, (0,))), (0,))


def _sc_select_digit(suffix, krem, suf_v):
    cnt = plsc.all_reduce_population_count(suffix >= krem)
    suf_v[...] = suffix
    above = plsc.load_gather(suf_v, [jnp.minimum(cnt, 15)])
    above = jnp.where(cnt >= 16, 0, above)
    return cnt, krem - above


def _sc_thr_body(keys_hbm, kv_hbm, thr_hbm, keys_v, hist_v,
                 suf_v, kv_v, thr_v, ba_v, bb_v):
    c = lax.axis_index("c")
    sb = lax.axis_index("s")
    wid = sb * 2 + c
    base = wid * RPT

    pltpu.sync_copy(kv_hbm, kv_v)
    ktest = lax.squeeze(lax.slice(kv_v[...], (0,), (1,)), dimensions=(0,))

    ones = jnp.ones((16,), jnp.int32)
    zeros16 = jnp.zeros((16,), jnp.int32)
    lanes = lax.iota(jnp.int32, 16)

    for b in range(RPT // RB):
        pltpu.sync_copy(keys_hbm.at[pl.ds((base + b * RB) * S, RB * S)],
                        keys_v)

        def row_fn(r, _):
            rowoff = r * S
            UNROLL = 16

            def zero_hist():
                for t in range(16):
                    hist_v[pl.ds(t * 16, 16)] = zeros16

            def select_byte(krem):
                h16 = plsc.load_gather(hist_v, [lanes * 16])
                for t in range(1, 16):
                    h16 = h16 + plsc.load_gather(hist_v, [lanes * 16 + t])
                cnt_hi, krem2 = _sc_select_digit(_sc_suffix(h16), krem, suf_v)
                fine = plsc.load_gather(hist_v, [(cnt_hi - 1) * 16 + lanes])
                cnt_lo, krem3 = _sc_select_digit(_sc_suffix(fine), krem2,
                                                 suf_v)
                return (cnt_hi - 1) * 16 + (cnt_lo - 1), krem3

            zero_hist()

            def scan1(j, _):
                for u in range(UNROLL):
                    x = keys_v[pl.ds(rowoff + (j * UNROLL + u) * 16, 16)]
                    plsc.addupdate_scatter(
                        hist_v, [(x >> 24) & jnp.int32(0xFF)], ones)
                return 0

            lax.fori_loop(0, NV // UNROLL, scan1, 0)
            d1, krem = select_byte(jnp.broadcast_to(ktest, (16,)))
            prefix = d1 << 24

            def comp1(j, off):
                for u in range(UNROLL):
                    x = keys_v[pl.ds(rowoff + (j * UNROLL + u) * 16, 16)]
                    act = ((x >> 24) & jnp.int32(0xFF)) == d1
                    pos = plsc.cumsum(act.astype(jnp.int32))
                    plsc.store_scatter(ba_v, [off + pos - 1], x, mask=act)
                    off = off + plsc.all_reduce_population_count(act)
                return off

            off = lax.fori_loop(0, NV // UNROLL, comp1, zeros16)
            n = lax.squeeze(lax.slice(off, (0,), (1,)), dimensions=(0,))

            for pi, shift in enumerate((16, 8, 0)):
                src = (ba_v, bb_v)[pi % 2]
                dst = (bb_v, ba_v)[pi % 2]
                nsp = jnp.broadcast_to(n, (16,))
                nv = (n + 15) >> 4
                zero_hist()

                def scanp(j, _, src=src, shift=shift, nsp=nsp):
                    x = src[pl.ds(j * 16, 16)]
                    tail = (j * 16 + lanes) < nsp
                    plsc.addupdate_scatter(
                        hist_v, [(x >> shift) & jnp.int32(0xFF)], ones,
                        mask=tail)
                    return 0

                lax.fori_loop(0, nv, scanp, 0)
                dsub, krem = select_byte(krem)
                prefix = prefix | (dsub << shift)

                if shift != 0:
                    def compp(j, off2, src=src, dst=dst, shift=shift,
                              nsp=nsp, dsub=dsub):
                        x = src[pl.ds(j * 16, 16)]
                        tail = (j * 16 + lanes) < nsp
                        act = tail & (((x >> shift) & jnp.int32(0xFF)) == dsub)
                        pos = plsc.cumsum(act.astype(jnp.int32))
                        plsc.store_scatter(dst, [off2 + pos - 1], x, mask=act)
                        return off2 + plsc.all_reduce_population_count(act)

                    off2 = lax.fori_loop(0, nv, compp, zeros16)
                    n = lax.squeeze(lax.slice(off2, (0,), (1,)),
                                    dimensions=(0,))

            idx = jnp.broadcast_to(b * RB + r, (16,)).astype(jnp.int32)
            plsc.store_scatter(thr_v, [idx], prefix, mask=lanes == 0)
            return 0

        lax.fori_loop(0, RB, row_fn, 0)

    pltpu.sync_copy(thr_v, thr_hbm.at[pl.ds(base, RPT)])


def _run_sc_thr(keys, kvec):
    mesh = plsc.VectorSubcoreMesh(core_axis_name="c", subcore_axis_name="s")
    fn = functools.partial(
        pl.kernel,
        mesh=mesh,
        compiler_params=pltpu.CompilerParams(needs_layout_passes=False),
        out_type=jax.ShapeDtypeStruct((S,), jnp.int32),
        scratch_types=[
            pltpu.VMEM((RB * S,), jnp.int32),
            pltpu.VMEM((256,), jnp.int32),
            pltpu.VMEM((16,), jnp.int32),
            pltpu.VMEM((16,), jnp.int32),
            pltpu.VMEM((RPT,), jnp.int32),
            pltpu.VMEM((S,), jnp.int32),
            pltpu.VMEM((S,), jnp.int32),
        ],
    )(_sc_thr_body)
    return fn(keys.reshape(-1), kvec)


def _attn_body(q_ref, k_ref, v_ref, keys_ref, thr_ref, out_ref):
    scale = 1.0 / (HEAD_DIM ** 0.5)
    keys = lax.bitcast_convert_type(keys_ref[...], jnp.uint32)
    thr = lax.bitcast_convert_type(thr_ref[...], jnp.uint32)
    add_mask = jnp.where(keys >= thr, 0.0, NEG_INF)
    logits = lax.dot_general(q_ref[...], k_ref[...],
                             dimension_numbers=((((1,), (1,))), ((), ())),
                             preferred_element_type=jnp.float32) * scale
    logits = logits + add_mask
    m = jnp.max(logits, axis=1, keepdims=True)
    p = jnp.exp(logits - m)
    z = jnp.sum(p, axis=1, keepdims=True)
    pv = lax.dot_general(p.astype(jnp.bfloat16), v_ref[...],
                         dimension_numbers=((((1,), (0,))), ((), ())),
                         preferred_element_type=jnp.float32)
    out_ref[...] = (pv * (1.0 / z)).astype(jnp.bfloat16)


def _run_attn(q, k, v, scores, thr):
    nq = q.shape[0] // BQ
    return pl.pallas_call(
        _attn_body,
        grid=(nq, N_HEADS),
        in_specs=[
            pl.BlockSpec((BQ, HEAD_DIM), lambda i, hh: (i, hh)),
            pl.BlockSpec((S, HEAD_DIM), lambda i, hh: (0, hh)),
            pl.BlockSpec((S, HEAD_DIM), lambda i, hh: (0, hh)),
            pl.BlockSpec((BQ, S), lambda i, hh: (i, 0)),
            pl.BlockSpec((BQ, 1), lambda i, hh: (i, 0)),
        ],
        out_specs=pl.BlockSpec((BQ, HEAD_DIM), lambda i, hh: (i, hh)),
        out_shape=jax.ShapeDtypeStruct((q.shape[0], D_MODEL), jnp.bfloat16),
    )(q, k, v, scores, thr)


def _oproj_body(x_ref, w_ref, out_ref):
    out_ref[...] = lax.dot_general(x_ref[...], w_ref[...],
                                   dimension_numbers=((((1,), (0,))), ((), ())),
                                   preferred_element_type=jnp.float32)


def _run_oproj(x, w):
    return pl.pallas_call(
        _oproj_body,
        grid=(N_QBLK,),
        in_specs=[
            pl.BlockSpec((BQ, D_MODEL), lambda i: (i, 0)),
            pl.BlockSpec((D_MODEL, D_MODEL), lambda i: (0, 0)),
        ],
        out_specs=pl.BlockSpec((BQ, D_MODEL), lambda i: (i, 0)),
        out_shape=jax.ShapeDtypeStruct((S, D_MODEL), jnp.float32),
    )(x, w)


def kernel(hidden_states, q_idx_W, q_idx_b, k_idx_W, k_idx_b, idx_weights,
           ctrl_W, ctrl_b, Wq, Wk, Wv, Wo):
    h16 = hidden_states[0].astype(jnp.bfloat16)

    w_idx = jnp.concatenate([q_idx_W, k_idx_W], axis=1).astype(jnp.bfloat16)
    w_qkv = jnp.concatenate([Wq, Wk, Wv], axis=1).astype(jnp.bfloat16)

    proj_idx, pooled = _run_proj_idx(h16, w_idx)
    qkv = _run_proj_qkv(h16, w_qkv)

    qi = proj_idx[:, :NIDX]
    ki = proj_idx[:, NIDX:]
    q = qkv[:, :D_MODEL]
    k = qkv[:, D_MODEL:2 * D_MODEL]
    v = qkv[:, 2 * D_MODEL:]

    keys, kvec = _run_scores(qi, ki, q_idx_b[None, :], k_idx_b[None, :],
                             idx_weights, pooled, ctrl_W, ctrl_b)

    thr = _run_sc_thr(keys, kvec[0, :16])
    attn = _run_attn(q, k, v, keys, thr[:, None])
    out = _run_oproj(attn, Wo.astype(jnp.bfloat16))
    return out[None]
